# Initial kernel scaffold; baseline (speedup 1.0000x reference)
import jax
import jax.numpy as jnp
from jax.experimental import pallas as pl


def kernel(rays_o, rays_d, weights):
    raise NotImplementedError("write your pallas kernel here")


# speedup vs baseline: 34.3876x; 34.3876x over previous
import functools

import jax
import jax.numpy as jnp
from jax import lax
from jax.experimental import pallas as pl
from jax.experimental.pallas import tpu as pltpu
from jax.experimental.pallas import tpu_sc as plsc

TINY = 1e-6
M = 128
N = 64
BATCH = 16384
NC, NS, L = 2, 16, 16
NW = NC * NS
RAYS_PER_W = BATCH // NW
G = 64
NBATCH = RAYS_PER_W // G
DELTA = 4.0 / 127.0


def _body(o_hbm, d_hbm, w_hbm, pts_hbm, z_hbm, s_hbm,
          w_v, o_v, d_v, cdf_v, pts_v, z_v):
    wid = lax.axis_index("s") * NC + lax.axis_index("c")
    iota = lax.iota(jnp.int32, L)
    iotaf = iota.astype(jnp.float32)

    def ray_body(r, carry):
        run = 0.0
        for k in range(M // L):
            ch = w_v[r, pl.ds(L * k, L)] + TINY
            cum = plsc.cumsum(ch) + run
            cdf_v[pl.ds(L * k, L)] = cum
            run = jnp.max(cum)
        total = run
        recip = 1.0 / jnp.full((L,), total, jnp.float32)

        rvec = jnp.full((L,), r, jnp.int32)
        ox = plsc.load_gather(o_v, [rvec, jnp.full((L,), 0, jnp.int32)])
        oy = plsc.load_gather(o_v, [rvec, jnp.full((L,), 1, jnp.int32)])
        oz = plsc.load_gather(o_v, [rvec, jnp.full((L,), 2, jnp.int32)])
        dx = plsc.load_gather(d_v, [rvec, jnp.full((L,), 0, jnp.int32)])
        dy = plsc.load_gather(d_v, [rvec, jnp.full((L,), 1, jnp.int32)])
        dz = plsc.load_gather(d_v, [rvec, jnp.full((L,), 2, jnp.int32)])

        for c in range(N // L):
            u = (iotaf + float(L * c)) * (1.0 / 63.0)
            U = u * total
            below = jnp.zeros((L,), jnp.int32)
            for step in (64, 32, 16, 8, 4, 2, 1):
                cand = below + step
                val = plsc.load_gather(cdf_v, [cand - 1])
                below = jnp.where(val <= U, cand, below)
            cBraw = plsc.load_gather(cdf_v, [jnp.maximum(below - 1, 0)])
            cB = jnp.where(below > 0, cBraw, 0.0)
            cA = plsc.load_gather(cdf_v, [below])
            denom = (cA - cB) * recip
            denom = jnp.where(denom < TINY, 1.0, denom)
            t = (u - cB * recip) / denom
            bf = below.astype(jnp.float32)
            blo = jnp.clip(bf - 0.5, 0.0, 127.0)
            bhi = jnp.minimum(bf + 0.5, 127.0)
            samples = 2.0 + blo * DELTA + t * ((bhi - blo) * DELTA + TINY)
            z_v[r, pl.ds(L * c, L)] = samples
            nidx = iota + L * c
            for comp, (o_s, d_s) in enumerate(((ox, dx), (oy, dy), (oz, dz))):
                cvec = jnp.full((L,), comp, jnp.int32)
                plsc.store_scatter(pts_v, [rvec, nidx, cvec], o_s + d_s * samples)
        return carry

    def batch_body(g, carry):
        base = wid * RAYS_PER_W + g * G
        pltpu.sync_copy(w_hbm.at[pl.ds(base, G)], w_v)
        pltpu.sync_copy(o_hbm.at[pl.ds(base, G)], o_v)
        pltpu.sync_copy(d_hbm.at[pl.ds(base, G)], d_v)
        lax.fori_loop(0, G, ray_body, 0, unroll=False)
        pltpu.sync_copy(pts_v, pts_hbm.at[pl.ds(base, G)])
        pltpu.sync_copy(z_v, z_hbm.at[pl.ds(base, G)])
        pltpu.sync_copy(z_v, s_hbm.at[pl.ds(base, G)])
        return carry

    lax.fori_loop(0, NBATCH, batch_body, 0, unroll=False)


@jax.jit
def kernel(rays_o, rays_d, weights):
    mesh = plsc.VectorSubcoreMesh(core_axis_name="c", subcore_axis_name="s")
    f = pl.kernel(
        _body,
        out_type=(
            jax.ShapeDtypeStruct((BATCH, N, 3), jnp.float32),
            jax.ShapeDtypeStruct((BATCH, N), jnp.float32),
            jax.ShapeDtypeStruct((BATCH, N), jnp.float32),
        ),
        mesh=mesh,
        compiler_params=pltpu.CompilerParams(
            needs_layout_passes=False, use_tc_tiling_on_sc=False),
        scratch_types=[
            pltpu.VMEM((G, M), jnp.float32),
            pltpu.VMEM((G, 3), jnp.float32),
            pltpu.VMEM((G, 3), jnp.float32),
            pltpu.VMEM((M,), jnp.float32),
            pltpu.VMEM((G, N, 3), jnp.float32),
            pltpu.VMEM((G, N), jnp.float32),
        ],
    )
    pts, z, s = f(rays_o, rays_d, weights)
    return (pts, z, s)
